# Initial kernel scaffold; baseline (speedup 1.0000x reference)
import jax
import jax.numpy as jnp
from jax.experimental import pallas as pl


def kernel(indices, tables, W1, b1, W2, b2, W3, b3):
    raise NotImplementedError("write your pallas kernel here")


# speedup vs baseline: 6.0353x; 6.0353x over previous
import functools

import jax
import jax.numpy as jnp
from jax import lax
from jax.experimental import pallas as pl
from jax.experimental.pallas import tpu as pltpu
from jax.experimental.pallas import tpu_sc as plsc

N_T = 26
VOCAB = 100000
D = 32
BATCH = 4096
L = 20

BAGS = N_T * BATCH
NW = 32
BAGS_PER_W = BAGS // NW
G = 64
CHUNKS = BAGS_PER_W // G
IDX_PER_CHUNK = G * L
GW = 128
K = IDX_PER_CHUNK // GW
IDX_ROWS_PER_W = BAGS_PER_W * L // GW


def _pooled_sc(idx3d, flat_tab):
    mesh = plsc.VectorSubcoreMesh(core_axis_name="c", subcore_axis_name="s")

    @functools.partial(
        pl.kernel,
        out_type=jax.ShapeDtypeStruct((BAGS, D), jnp.float32),
        mesh=mesh,
        scratch_types=[
            pltpu.VMEM((K, GW), jnp.int32),
            pltpu.VMEM((IDX_PER_CHUNK, D), jnp.float32),
            pltpu.VMEM((G, D), jnp.float32),
            pltpu.SemaphoreType.DMA,
        ],
        compiler_params=pltpu.CompilerParams(use_tc_tiling_on_sc=False),
    )
    def k(idx_hbm, tab_hbm, out_hbm, idx_v, rows_v, out_v, sem):
        wid = lax.axis_index("s") * 2 + lax.axis_index("c")
        bag_base = wid * BAGS_PER_W

        @pl.loop(0, CHUNKS)
        def _(c):
            bag0 = bag_base + c * G
            pltpu.sync_copy(idx_hbm.at[wid * CHUNKS + c], idx_v)
            copies = []
            for j in range(K):
                copies.append(
                    pltpu.async_copy(
                        tab_hbm.at[idx_v.at[j]],
                        rows_v.at[pl.ds(j * GW, GW)],
                        sem,
                    )
                )
            for cp in copies:
                cp.wait()

            @pl.loop(0, G)
            def _(g):
                r0 = g * L
                a0 = rows_v[r0, pl.ds(0, 16)]
                a1 = rows_v[r0, pl.ds(16, 16)]
                for step in range(1, L):
                    a0 = a0 + rows_v[r0 + step, pl.ds(0, 16)]
                    a1 = a1 + rows_v[r0 + step, pl.ds(16, 16)]
                out_v[g, pl.ds(0, 16)] = a0
                out_v[g, pl.ds(16, 16)] = a1

            pltpu.sync_copy(out_v, out_hbm.at[pl.ds(bag0, G)])

    return k(idx3d, flat_tab)


BLK = 2048


def _mlp_tc(x, wt1, b1, wt2, b2, wt3, b3):
    def body(x_ref, w1_ref, b1_ref, w2_ref, b2_ref, w3_ref, b3_ref, o_ref):
        dn = (((1,), (0,)), ((), ()))
        h = x_ref[...]
        h = lax.dot_general(h, w1_ref[...], dn,
                            precision=lax.Precision.HIGHEST) + b1_ref[...]
        h = lax.dot_general(h, w2_ref[...], dn,
                            precision=lax.Precision.HIGHEST) + b2_ref[...]
        h = lax.dot_general(h, w3_ref[...], dn,
                            precision=lax.Precision.HIGHEST) + b3_ref[...]
        o_ref[...] = h

    wspec = pl.BlockSpec((D, D), lambda i: (0, 0))
    bspec = pl.BlockSpec((1, D), lambda i: (0, 0))
    return pl.pallas_call(
        body,
        grid=(BAGS // BLK,),
        in_specs=[pl.BlockSpec((BLK, D), lambda i: (i, 0)),
                  wspec, bspec, wspec, bspec, wspec, bspec],
        out_specs=pl.BlockSpec((BLK, D), lambda i: (i, 0)),
        out_shape=jax.ShapeDtypeStruct((BAGS, D), jnp.float32),
    )(x, wt1, b1, wt2, b2, wt3, b3)


def kernel(indices, tables, W1, b1, W2, b2, W3, b3):
    offs = (jnp.arange(N_T, dtype=jnp.int32) * VOCAB)[:, None, None]
    idx3d = (indices.astype(jnp.int32) + offs).reshape(NW * CHUNKS, K, GW)
    flat_tab = tables.reshape(N_T * VOCAB, D)
    pooled = _pooled_sc(idx3d, flat_tab)
    return _mlp_tc(pooled, W1.T, b1.reshape(1, D), W2.T, b2.reshape(1, D),
                   W3.T, b3.reshape(1, D))
